# Initial kernel scaffold; baseline (speedup 1.0000x reference)
import jax
import jax.numpy as jnp
from jax.experimental import pallas as pl


def kernel(inputs):
    raise NotImplementedError("write your pallas kernel here")


# speedup vs baseline: 25.3434x; 25.3434x over previous
import numpy as np
import jax
import jax.numpy as jnp
from jax.experimental import pallas as pl

TOPK = 64
E = 64
CBLK = 256


def _cx(x, j, desc_full):
    e, g, c = x.shape
    xr = x.reshape(e // (2 * j), 2, j, g, c)
    m = desc_full.reshape(e // (2 * j), 2, j, g, 1)[:, 0]
    a, b = xr[:, 0], xr[:, 1]
    mx = jnp.maximum(a, b)
    mn = jnp.minimum(a, b)
    first = jnp.where(m, mx, mn)
    second = jnp.where(m, mn, mx)
    return jnp.concatenate([first[:, None], second[:, None]], axis=1).reshape(e, g, c)


def _topk_block(x):
    l, c = x.shape
    g = l // E
    x = x.reshape(E, g, c)

    def iotas(g):
        i = jax.lax.broadcasted_iota(jnp.int32, (E, g, 1), 0)
        gi = jax.lax.broadcasted_iota(jnp.int32, (E, g, 1), 1)
        return i, gi

    i, gi = iotas(g)
    col_desc = gi < g // 2
    k = 2
    while k <= E:
        bit = (i & k) == 0
        m = bit == col_desc
        j = k // 2
        while j >= 1:
            x = _cx(x, j, m)
            j //= 2
        k *= 2

    while g > 1:
        gh = g // 2
        y = jnp.maximum(x[:, :gh], x[:, gh:])
        _, gi = iotas(gh)
        ndesc = gh // 2 if gh > 1 else 1
        m = gi < ndesc
        j = E // 2
        while j >= 1:
            y = _cx(y, j, m)
            j //= 2
        x = y
        g = gh
    return x[:, 0, :]


def _topk_kernel(x_ref, o_ref):
    o_ref[0] = _topk_block(x_ref[0])


def kernel(inputs):
    b, l, c = inputs.shape
    return pl.pallas_call(
        _topk_kernel,
        grid=(b, c // CBLK),
        in_specs=[pl.BlockSpec((1, l, CBLK), lambda i, j: (i, 0, j))],
        out_specs=pl.BlockSpec((1, TOPK, CBLK), lambda i, j: (i, 0, j)),
        out_shape=jax.ShapeDtypeStruct((b, TOPK, c), inputs.dtype),
    )(inputs)


# speedup vs baseline: 27.5877x; 1.0886x over previous
import numpy as np
import jax
import jax.numpy as jnp
from jax.experimental import pallas as pl

TOPK = 64
E = 64
CBLK = 256


def _ce(x, j, desc):
    m, n, g, c = x.shape
    xr = x.reshape(m, n // (2 * j), 2, j, g, c)
    a, b = xr[:, :, 0], xr[:, :, 1]
    hi = jnp.maximum(a, b)
    lo = jnp.minimum(a, b)
    f, s = (hi, lo) if desc else (lo, hi)
    return jnp.concatenate([f[:, :, None], s[:, :, None]], axis=2).reshape(m, n, g, c)


def _sort_cols(x, desc):
    e, g, c = x.shape
    k = 2
    while k < e:
        runs = x.reshape(e // (2 * k), 2, k, g, c)
        de, asc = runs[:, 0], runs[:, 1]
        j = k // 2
        while j >= 1:
            de = _ce(de, j, desc)
            asc = _ce(asc, j, not desc)
            j //= 2
        x = jnp.concatenate([de[:, None], asc[:, None]], axis=1).reshape(e, g, c)
        k *= 2
    y = x.reshape(1, e, g, c)
    j = e // 2
    while j >= 1:
        y = _ce(y, j, desc)
        j //= 2
    return y.reshape(e, g, c)


def _cleanup(y, desc):
    e, g, c = y.shape
    z = y.reshape(1, e, g, c)
    j = e // 2
    while j >= 1:
        z = _ce(z, j, desc)
        j //= 2
    return z.reshape(e, g, c)


def _topk_block(x):
    l, c = x.shape
    g = l // E
    x = x.reshape(E, g, c)
    left = _sort_cols(x[:, : g // 2], True)
    right = _sort_cols(x[:, g // 2 :], False)
    while True:
        y = jnp.maximum(left, right)
        g //= 2
        if g == 1:
            return _cleanup(y, True)[:, 0, :]
        left = _cleanup(y[:, : g // 2], True)
        right = _cleanup(y[:, g // 2 :], False)


def _topk_kernel(x_ref, o_ref):
    o_ref[0] = _topk_block(x_ref[0])


def kernel(inputs):
    b, l, c = inputs.shape
    return pl.pallas_call(
        _topk_kernel,
        grid=(b, c // CBLK),
        in_specs=[pl.BlockSpec((1, l, CBLK), lambda i, j: (i, 0, j))],
        out_specs=pl.BlockSpec((1, TOPK, CBLK), lambda i, j: (i, 0, j)),
        out_shape=jax.ShapeDtypeStruct((b, TOPK, c), inputs.dtype),
    )(inputs)


# speedup vs baseline: 34.3153x; 1.2439x over previous
import numpy as np
import jax
import jax.numpy as jnp
from jax.experimental import pallas as pl

TOPK = 64
E = 64
CBLK = 256


def _ce(p, i, j, desc):
    a, b = p[i], p[j]
    hi = jnp.maximum(a, b)
    lo = jnp.minimum(a, b)
    p[i], p[j] = (hi, lo) if desc else (lo, hi)


def _sort_pieces(p, desc):
    k = 2
    while k <= E:
        j = k // 2
        while j >= 1:
            for i in range(E):
                if i & j == 0:
                    _ce(p, i, i + j, ((i & k) == 0) == desc)
            j //= 2
        k *= 2


def _cleanup(p, desc):
    j = E // 2
    while j >= 1:
        for i in range(E):
            if i & j == 0:
                _ce(p, i, i + j, desc)
        j //= 2


def _topk_kernel(x_ref, o_ref):
    g = x_ref.shape[1] // E
    gh = g // 2
    pl_ = [x_ref[0, i * g : i * g + gh, :] for i in range(E)]
    pr_ = [x_ref[0, i * g + gh : (i + 1) * g, :] for i in range(E)]
    _sort_pieces(pl_, True)
    _sort_pieces(pr_, False)
    g = gh
    while True:
        y = [jnp.maximum(a, b) for a, b in zip(pl_, pr_)]
        if g == 1:
            _cleanup(y, True)
            for i in range(E):
                o_ref[0, i, :] = y[i][0, :]
            return
        pl_ = [t[: g // 2] for t in y]
        pr_ = [t[g // 2 :] for t in y]
        _cleanup(pl_, True)
        _cleanup(pr_, False)
        g //= 2


def kernel(inputs):
    b, l, c = inputs.shape
    return pl.pallas_call(
        _topk_kernel,
        grid=(b, c // CBLK),
        in_specs=[pl.BlockSpec((1, l, CBLK), lambda i, j: (i, 0, j))],
        out_specs=pl.BlockSpec((1, TOPK, CBLK), lambda i, j: (i, 0, j)),
        out_shape=jax.ShapeDtypeStruct((b, TOPK, c), inputs.dtype),
    )(inputs)
